# baseline (speedup 1.0000x reference)
import jax
import jax.numpy as jnp
from jax import lax
from jax.experimental import pallas as pl
from jax.experimental.pallas import tpu as pltpu
from jax.experimental.pallas import tpu_sc as plsc

_NC = 2
_NS = 16
_NW = _NC * _NS
_LANES = 16

_STRIP = 128
_STRIPS_PER_CHUNK = 13
_CHUNK = _STRIP * _STRIPS_PER_CHUNK


def _build_gather(N, H, D, n_off_pad):
    assert N % _NW == 0
    per_w = N // _NW
    assert per_w % _CHUNK == 0
    chunks = per_w // _CHUNK

    mesh = plsc.VectorSubcoreMesh(core_axis_name="c", subcore_axis_name="s")

    @pl.kernel(
        out_type=jax.ShapeDtypeStruct((N, D), jnp.float32),
        mesh=mesh,
        compiler_params=pltpu.CompilerParams(
            needs_layout_passes=False, use_tc_tiling_on_sc=False),
        scratch_types=[
            pltpu.VMEM((n_off_pad,), jnp.int32),
            pltpu.VMEM((2, _CHUNK), jnp.int32),
            pltpu.VMEM((2, _CHUNK, D), jnp.float32),
            pltpu.SemaphoreType.DMA,
            pltpu.SemaphoreType.DMA,
            pltpu.SemaphoreType.DMA,
            pltpu.SemaphoreType.DMA,
        ],
    )
    def gather_kernel(ids_hbm, offs_hbm, table_hbm, out_hbm,
                      offs_v, idx_v, rows_v, gsem_a, gsem_b, osem_a, osem_b):
        wid = lax.axis_index("s") * _NC + lax.axis_index("c")
        base = wid * per_w

        pltpu.sync_copy(offs_hbm, offs_v)
        iota = lax.broadcasted_iota(jnp.int32, (_LANES,), 0)

        gsems = [gsem_a, gsem_b]
        osems = [osem_a, osem_b]
        pending_gather = [None, None]
        pending_out = [None, None]

        def issue(g):
            b = g % 2
            if pending_out[b] is not None:
                pending_out[b].wait()
                pending_out[b] = None
            start = base + g * _CHUNK
            pltpu.sync_copy(ids_hbm.at[pl.ds(start, _CHUNK)], idx_v.at[b])

            def add_offsets(j, carry):
                lane0 = pl.multiple_of(j * _LANES, _LANES)
                pos = start + j * _LANES + iota
                off = plsc.load_gather(offs_v, [lax.rem(pos, H)])
                idx_v[b, pl.ds(lane0, _LANES)] = (
                    idx_v[b, pl.ds(lane0, _LANES)] + off)
                return carry

            lax.fori_loop(0, _CHUNK // _LANES, add_offsets, 0)

            copies = []
            for k in range(_STRIPS_PER_CHUNK):
                c = pltpu.async_copy(
                    table_hbm.at[idx_v.at[b, pl.ds(k * _STRIP, _STRIP)]],
                    rows_v.at[b, pl.ds(k * _STRIP, _STRIP)],
                    gsems[b])
                copies.append(c)
            pending_gather[b] = (g, copies)

        def drain(b):
            g, copies = pending_gather[b]
            for c in copies:
                c.wait()
            pending_gather[b] = None
            pending_out[b] = pltpu.async_copy(
                rows_v.at[b],
                out_hbm.at[pl.ds(base + g * _CHUNK, _CHUNK)],
                osems[b])

        issue(0)
        for g in range(1, chunks):
            issue(g)
            drain((g - 1) % 2)
        drain((chunks - 1) % 2)
        for b in range(2):
            if pending_out[b] is not None:
                pending_out[b].wait()

    return gather_kernel


def kernel(head_ids, offsets, table):
    B, S, H = head_ids.shape
    V, D = table.shape
    N = B * S * H
    n_off_pad = 128
    ids = head_ids.reshape(N).astype(jnp.int32)
    offs = jnp.zeros((n_off_pad,), jnp.int32).at[:H].set(
        offsets.astype(jnp.int32))
    out = _build_gather(N, H, D, n_off_pad)(ids, offs, table)
    return out.reshape(B, S, H, D)
